# baseline (speedup 1.0000x reference)
import functools

import jax
import jax.numpy as jnp
from jax import lax
from jax.experimental import pallas as pl
from jax.experimental.pallas import tpu as pltpu
from jax.experimental.pallas import tpu_sc as plsc

_B, _L, _C = 16, 4096, 256
_HALF = _L // 2
_CH = 128
_NCH = _HALF // _CH
_NCC = _C // 16


def _sc_reduce_body(x_hbm, m_hbm, psum, pmax, pcnt,
                    xbuf, mbuf, accbuf, maxbuf, cntbuf, xsem, msem):
    ci = lax.axis_index("c")
    si = lax.axis_index("s")
    row0 = si * _L + ci * _HALF

    def load(ch, slot):
        r = row0 + ch * _CH
        pltpu.make_async_copy(x_hbm.at[pl.ds(r, _CH)], xbuf.at[slot],
                              xsem.at[slot]).start()
        pltpu.make_async_copy(m_hbm.at[pl.ds(r, _CH)], mbuf.at[slot],
                              msem.at[slot]).start()

    def wait(ch, slot):
        r = row0 + ch * _CH
        pltpu.make_async_copy(x_hbm.at[pl.ds(r, _CH)], xbuf.at[slot],
                              xsem.at[slot]).wait()
        pltpu.make_async_copy(m_hbm.at[pl.ds(r, _CH)], mbuf.at[slot],
                              msem.at[slot]).wait()

    sums = [jnp.zeros((16,), jnp.float32) for _ in range(_NCC)]
    maxs = [jnp.full((16,), -1e30, jnp.float32) for _ in range(_NCC)]
    cnt = jnp.zeros((16,), jnp.float32)

    load(0, 0)
    carry = (*sums, *maxs, cnt)
    for ch in range(_NCH):
        slot = ch % 2
        if ch + 1 < _NCH:
            load(ch + 1, 1 - slot)
        wait(ch, slot)

        def tok(t, carry, slot=slot):
            sums = list(carry[:_NCC])
            maxs = list(carry[_NCC:2 * _NCC])
            cnt = carry[2 * _NCC]
            mvec = mbuf[slot, t]
            minf = (mvec - 1.0) * jnp.float32(1e30)
            for cc in range(_NCC):
                xv = xbuf[slot, t, pl.ds(cc * 16, 16)]
                sums[cc] = sums[cc] + xv * mvec
                maxs[cc] = jnp.maximum(maxs[cc], xv + minf)
            return (*sums, *maxs, cnt + mvec)

        carry = lax.fori_loop(0, _CH, tok, carry)

    for cc in range(_NCC):
        accbuf[pl.ds(cc * 16, 16)] = carry[cc]
        maxbuf[pl.ds(cc * 16, 16)] = carry[_NCC + cc]
    cntbuf[...] = carry[2 * _NCC]

    pltpu.sync_copy(accbuf, psum.at[ci, si])
    pltpu.sync_copy(maxbuf, pmax.at[ci, si])
    pltpu.sync_copy(cntbuf, pcnt.at[ci, si])


_sc_reduce = functools.partial(
    pl.kernel,
    mesh=plsc.VectorSubcoreMesh(core_axis_name="c", subcore_axis_name="s"),
    out_type=[
        jax.ShapeDtypeStruct((2, _B, _C), jnp.float32),
        jax.ShapeDtypeStruct((2, _B, _C), jnp.float32),
        jax.ShapeDtypeStruct((2, _B, 16), jnp.float32),
    ],
    scratch_types=[
        pltpu.VMEM((2, _CH, _C), jnp.float32),
        pltpu.VMEM((2, _CH, 16), jnp.float32),
        pltpu.VMEM((_C,), jnp.float32),
        pltpu.VMEM((_C,), jnp.float32),
        pltpu.VMEM((16,), jnp.float32),
        pltpu.SemaphoreType.DMA((2,)),
        pltpu.SemaphoreType.DMA((2,)),
    ],
)(_sc_reduce_body)


def _gate_scale_body(psum_ref, pmax_ref, pcnt_ref, w0_ref, w1_ref,
                     x_hbm, o_hbm, xbuf, obuf, lsem, ssem):
    B, L, C = x_hbm.shape
    sums = psum_ref[0] + psum_ref[1]
    mx = jnp.maximum(pmax_ref[0], pmax_ref[1])
    cnt = jnp.sum(jnp.sum(pcnt_ref[...], axis=0), axis=1)
    mean = sums / jnp.maximum(cnt, 1.0)[:, None]
    w0 = w0_ref[...]
    w1 = w1_ref[...]

    def mlp(v):
        h = lax.dot_general(v, w0, (((1,), (1,)), ((), ())),
                            preferred_element_type=jnp.float32)
        h = jnp.maximum(h, 0.0)
        return lax.dot_general(h, w1, (((1,), (1,)), ((), ())),
                               preferred_element_type=jnp.float32)

    a = jax.nn.sigmoid(mlp(mean) + mlp(mx))

    def start_load(b, slot):
        pltpu.make_async_copy(x_hbm.at[b], xbuf.at[slot], lsem.at[slot]).start()

    def wait_load(b, slot):
        pltpu.make_async_copy(x_hbm.at[b], xbuf.at[slot], lsem.at[slot]).wait()

    def start_store(b, slot):
        pltpu.make_async_copy(obuf.at[slot], o_hbm.at[b], ssem.at[slot]).start()

    def wait_store(b, slot):
        pltpu.make_async_copy(obuf.at[slot], o_hbm.at[b], ssem.at[slot]).wait()

    start_load(0, 0)
    start_load(1, 1)
    for b in range(B):
        if b + 2 < B:
            start_load(b + 2, (b + 2) % 4)
        wait_load(b, b % 4)
        if b >= 2:
            wait_store(b - 2, b % 2)
        obuf[b % 2] = xbuf[b % 4] * a[b:b + 1, :]
        start_store(b, b % 2)
    wait_store(B - 2, 0)
    wait_store(B - 1, 1)


def kernel(x, attention_mask, W0, W1):
    B, L, C = x.shape
    x2 = x.reshape(B * L, C)
    m16 = jnp.broadcast_to(
        attention_mask.astype(jnp.float32).reshape(B * L, 1), (B * L, 16))
    psum, pmax, pcnt = _sc_reduce(x2, m16)
    return pl.pallas_call(
        _gate_scale_body,
        in_specs=[
            pl.BlockSpec(memory_space=pltpu.MemorySpace.VMEM),
            pl.BlockSpec(memory_space=pltpu.MemorySpace.VMEM),
            pl.BlockSpec(memory_space=pltpu.MemorySpace.VMEM),
            pl.BlockSpec(memory_space=pltpu.MemorySpace.VMEM),
            pl.BlockSpec(memory_space=pltpu.MemorySpace.VMEM),
            pl.BlockSpec(memory_space=pl.ANY),
        ],
        out_specs=pl.BlockSpec(memory_space=pl.ANY),
        out_shape=jax.ShapeDtypeStruct(x.shape, x.dtype),
        scratch_shapes=[
            pltpu.VMEM((4, L, C), jnp.float32),
            pltpu.VMEM((2, L, C), jnp.float32),
            pltpu.SemaphoreType.DMA((4,)),
            pltpu.SemaphoreType.DMA((2,)),
        ],
    )(psum, pmax, pcnt, W0, W1, x)


# speedup vs baseline: 2.8374x; 2.8374x over previous
import jax
import jax.numpy as jnp
from jax import lax
from jax.experimental import pallas as pl
from jax.experimental.pallas import tpu as pltpu


def _body(mw_ref, w0_ref, w1_ref, x_hbm, o_hbm, xbuf, obuf, lsem, ssem):
    B, L, C = x_hbm.shape
    w0 = w0_ref[...]
    w1 = w1_ref[...]

    def start_load(b, slot):
        pltpu.make_async_copy(x_hbm.at[b], xbuf.at[slot], lsem.at[slot]).start()

    def wait_load(b, slot):
        pltpu.make_async_copy(x_hbm.at[b], xbuf.at[slot], lsem.at[slot]).wait()

    def start_store(b, slot):
        pltpu.make_async_copy(obuf.at[slot], o_hbm.at[b], ssem.at[slot]).start()

    def wait_store(b, slot):
        pltpu.make_async_copy(obuf.at[slot], o_hbm.at[b], ssem.at[slot]).wait()

    def compute(b, slot, oslot):
        xb = xbuf[slot]
        mrow = mw_ref[b]
        sums = lax.dot_general(mrow, xb, (((1,), (0,)), ((), ())),
                               preferred_element_type=jnp.float32)
        cnt = jnp.sum(mrow)
        mean = sums / jnp.maximum(cnt, 1.0)
        mcol = mrow.reshape(L, 1)
        neg = jnp.where(mcol > 0.0, xb, jnp.float32(-1e30))
        mx = jnp.max(neg, axis=0, keepdims=True)

        def mlp(v):
            h = lax.dot_general(v, w0, (((1,), (1,)), ((), ())),
                                preferred_element_type=jnp.float32)
            h = jnp.maximum(h, 0.0)
            return lax.dot_general(h, w1, (((1,), (1,)), ((), ())),
                                   preferred_element_type=jnp.float32)

        a = jax.nn.sigmoid(mlp(mean) + mlp(mx))
        obuf[oslot] = xb * a

    start_load(0, 0)
    start_load(1, 1)
    start_load(2, 2)
    for b in range(B):
        if b + 3 < B:
            start_load(b + 3, (b + 3) % 4)
        wait_load(b, b % 4)
        if b >= 2:
            wait_store(b - 2, b % 2)
        compute(b, b % 4, b % 2)
        start_store(b, b % 2)
    wait_store(B - 2, 0)
    wait_store(B - 1, 1)


def kernel(x, attention_mask, W0, W1):
    B, L, C = x.shape
    mw = attention_mask.astype(jnp.float32).reshape(B, 1, L)
    return pl.pallas_call(
        _body,
        in_specs=[
            pl.BlockSpec(memory_space=pltpu.MemorySpace.VMEM),
            pl.BlockSpec(memory_space=pltpu.MemorySpace.VMEM),
            pl.BlockSpec(memory_space=pltpu.MemorySpace.VMEM),
            pl.BlockSpec(memory_space=pl.ANY),
        ],
        out_specs=pl.BlockSpec(memory_space=pl.ANY),
        out_shape=jax.ShapeDtypeStruct(x.shape, x.dtype),
        scratch_shapes=[
            pltpu.VMEM((4, L, C), jnp.float32),
            pltpu.VMEM((2, L, C), jnp.float32),
            pltpu.SemaphoreType.DMA((4,)),
            pltpu.SemaphoreType.DMA((2,)),
        ],
    )(mw, W0, W1, x)
